# Initial kernel scaffold; baseline (speedup 1.0000x reference)
import jax
import jax.numpy as jnp
from jax.experimental import pallas as pl


def kernel(x, edge_index, init_w1, w1, rw1, b1, g1, be1, init_w2, w2, rw2, b2, g2, be2):
    raise NotImplementedError("write your pallas kernel here")


# speedup vs baseline: 15.0478x; 15.0478x over previous
import functools

import jax
import jax.numpy as jnp
from jax import lax
from jax.experimental import pallas as pl
from jax.experimental.pallas import tpu as pltpu
from jax.experimental.pallas import tpu_sc as plsc

N = 10000
E = 160000
F = 256
K = 3
EPS = 1e-5

NSC = 2
NTILE = 16
EB = 128
E_PAD = 163840
NACC = N + 112
STR = NACC // NTILE

RB = 400
NRB = N // RB


def _make_spmm(C):
    mesh = plsc.VectorSubcoreMesh(core_axis_name="c", subcore_axis_name="s")
    epw = E_PAD // NSC // NTILE
    nb = epw // EB

    @functools.partial(
        pl.kernel,
        mesh=mesh,
        out_type=jax.ShapeDtypeStruct((C, NSC, NACC, 128), jnp.float32),
        scratch_types=[
            pltpu.VMEM((EB,), jnp.int32),
            pltpu.VMEM((EB,), jnp.int32),
            pltpu.VMEM((EB, 128), jnp.float32),
            pltpu.VMEM_SHARED((NACC, 128), jnp.float32),
            pltpu.SemaphoreType.DMA,
        ],
    )
    def spmm(xs_hbm, rowp_hbm, colp_hbm, zeros_hbm, out_hbm,
             rowbuf, colbuf, rows, acc, sem):
        core = lax.axis_index("c")
        sub = lax.axis_index("s")
        base = core * (E_PAD // NSC) + sub * epw
        stripe = sub * STR
        for c in range(C):
            pltpu.sync_copy(zeros_hbm.at[pl.ds(stripe, STR)],
                            acc.at[pl.ds(stripe, STR)])
            plsc.subcore_barrier()

            def body(b, carry):
                off = base + b * EB
                pltpu.sync_copy(rowp_hbm.at[pl.ds(off, EB)], rowbuf)
                pltpu.sync_copy(colp_hbm.at[pl.ds(off, EB)], colbuf)
                pltpu.async_copy(xs_hbm.at[c].at[rowbuf], rows, sem).wait()
                pltpu.sync_copy(rows, acc.at[colbuf], add=True)
                return carry

            lax.fori_loop(0, nb, body, 0)
            plsc.subcore_barrier()
            pltpu.sync_copy(acc.at[pl.ds(stripe, STR)],
                            out_hbm.at[c].at[core].at[pl.ds(stripe, STR)])

    return spmm


def _k1_body(degp_ref, x_ref, dis_ref, xs_ref):
    deg = degp_ref[0, 0] + degp_ref[0, 1]
    dis = jnp.where(deg > 0.0, lax.rsqrt(jnp.maximum(deg, 1.0)), 0.0)
    dis_ref[...] = dis
    x = x_ref[...]
    for c in range(2):
        xs_ref[c] = dis * x[:, c * 128:(c + 1) * 128]


def _k1(degp, x):
    return pl.pallas_call(
        _k1_body,
        grid=(NRB,),
        in_specs=[
            pl.BlockSpec((1, NSC, RB, 128), lambda i: (0, 0, i, 0)),
            pl.BlockSpec((RB, F), lambda i: (i, 0)),
        ],
        out_specs=[
            pl.BlockSpec((RB, 128), lambda i: (i, 0)),
            pl.BlockSpec((2, RB, 128), lambda i: (0, i, 0)),
        ],
        out_shape=[
            jax.ShapeDtypeStruct((N, 128), jnp.float32),
            jax.ShapeDtypeStruct((2, N, 128), jnp.float32),
        ],
    )(degp, x)


def _kb_body(gp_ref, x_ref, dis_ref, iw_ref, rw_ref, bias_ref, o_ref):
    g = jnp.concatenate(
        [gp_ref[c, 0] + gp_ref[c, 1] for c in range(2)], axis=1)
    d1 = dis_ref[:, :1]
    t = jnp.dot(g, iw_ref[...], preferred_element_type=jnp.float32)
    r = jnp.dot(x_ref[...], rw_ref[...], preferred_element_type=jnp.float32)
    o1 = jax.nn.relu(d1 * t + r + bias_ref[0:1, :])
    o1s = d1 * o1
    for c in range(2 * K):
        o_ref[c] = o1s[:, c * 128:(c + 1) * 128]


def _kb(gp, x, dis, iw, rw, bias):
    return pl.pallas_call(
        _kb_body,
        grid=(NRB,),
        in_specs=[
            pl.BlockSpec((2, NSC, RB, 128), lambda i: (0, 0, i, 0)),
            pl.BlockSpec((RB, F), lambda i: (i, 0)),
            pl.BlockSpec((RB, 128), lambda i: (i, 0)),
            pl.BlockSpec((F, K * F), lambda i: (0, 0)),
            pl.BlockSpec((F, K * F), lambda i: (0, 0)),
            pl.BlockSpec((8, K * F), lambda i: (0, 0)),
        ],
        out_specs=pl.BlockSpec((2 * K, RB, 128), lambda i: (0, i, 0)),
        out_shape=jax.ShapeDtypeStruct((2 * K, N, 128), jnp.float32),
    )(gp, x, dis, iw, rw, bias)


def _kc1_body(gp_ref, x_ref, dis_ref, w_ref, rw_ref, bias_ref, h_ref, st_ref):
    i = pl.program_id(0)
    d1 = dis_ref[:, :1]
    x = x_ref[...]
    acc = jnp.zeros((RB, F), jnp.float32)
    for k in range(K):
        g = jnp.concatenate(
            [gp_ref[2 * k + c, 0] + gp_ref[2 * k + c, 1] for c in range(2)],
            axis=1)
        t = jnp.dot(g, w_ref[k], preferred_element_type=jnp.float32)
        r = jnp.dot(x, rw_ref[k], preferred_element_type=jnp.float32)
        acc = acc + jax.nn.relu(d1 * t + r + bias_ref[k, 0:1, :])
    h = acc * (1.0 / K)
    h_ref[...] = h
    s1 = jnp.sum(h, axis=0, keepdims=True)
    s2 = jnp.sum(h * h, axis=0, keepdims=True)
    st = jnp.concatenate([s1, s2, jnp.zeros((6, F), jnp.float32)], axis=0)

    @pl.when(i == 0)
    def _():
        st_ref[...] = jnp.zeros_like(st_ref)

    st_ref[...] += st


def _kc1(gp, x, dis, w, rw, bias):
    return pl.pallas_call(
        _kc1_body,
        grid=(NRB,),
        in_specs=[
            pl.BlockSpec((2 * K, NSC, RB, 128), lambda i: (0, 0, i, 0)),
            pl.BlockSpec((RB, F), lambda i: (i, 0)),
            pl.BlockSpec((RB, 128), lambda i: (i, 0)),
            pl.BlockSpec((K, F, F), lambda i: (0, 0, 0)),
            pl.BlockSpec((K, F, F), lambda i: (0, 0, 0)),
            pl.BlockSpec((K, 8, F), lambda i: (0, 0, 0)),
        ],
        out_specs=[
            pl.BlockSpec((RB, F), lambda i: (i, 0)),
            pl.BlockSpec((8, F), lambda i: (0, 0)),
        ],
        out_shape=[
            jax.ShapeDtypeStruct((N, F), jnp.float32),
            jax.ShapeDtypeStruct((8, F), jnp.float32),
        ],
    )(gp, x, dis, w, rw, bias)


def _kc2_mid_body(h_ref, st_ref, gm_ref, bt_ref, dis_ref, y_ref, ys_ref):
    h = h_ref[...]
    m = st_ref[0:1, :] * (1.0 / N)
    v = st_ref[1:2, :] * (1.0 / N) - m * m
    bn = gm_ref[0:1, :] * (h - m) * lax.rsqrt(v + EPS) + bt_ref[0:1, :]
    y = jax.nn.relu(bn)
    y_ref[...] = y
    ys = dis_ref[:, :1] * y
    for c in range(2):
        ys_ref[c] = ys[:, c * 128:(c + 1) * 128]


def _kc2_mid(h, st, gm, bt, dis):
    return pl.pallas_call(
        _kc2_mid_body,
        grid=(NRB,),
        in_specs=[
            pl.BlockSpec((RB, F), lambda i: (i, 0)),
            pl.BlockSpec((8, F), lambda i: (0, 0)),
            pl.BlockSpec((8, F), lambda i: (0, 0)),
            pl.BlockSpec((8, F), lambda i: (0, 0)),
            pl.BlockSpec((RB, 128), lambda i: (i, 0)),
        ],
        out_specs=[
            pl.BlockSpec((RB, F), lambda i: (i, 0)),
            pl.BlockSpec((2, RB, 128), lambda i: (0, i, 0)),
        ],
        out_shape=[
            jax.ShapeDtypeStruct((N, F), jnp.float32),
            jax.ShapeDtypeStruct((2, N, 128), jnp.float32),
        ],
    )(h, st, gm, bt, dis)


def _kc2_final_body(h_ref, st_ref, gm_ref, bt_ref, y_ref):
    h = h_ref[...]
    m = st_ref[0:1, :] * (1.0 / N)
    v = st_ref[1:2, :] * (1.0 / N) - m * m
    bn = gm_ref[0:1, :] * (h - m) * lax.rsqrt(v + EPS) + bt_ref[0:1, :]
    y_ref[...] = jax.nn.relu(bn)


def _kc2_final(h, st, gm, bt):
    return pl.pallas_call(
        _kc2_final_body,
        grid=(NRB,),
        in_specs=[
            pl.BlockSpec((RB, F), lambda i: (i, 0)),
            pl.BlockSpec((8, F), lambda i: (0, 0)),
            pl.BlockSpec((8, F), lambda i: (0, 0)),
            pl.BlockSpec((8, F), lambda i: (0, 0)),
        ],
        out_specs=pl.BlockSpec((RB, F), lambda i: (i, 0)),
        out_shape=jax.ShapeDtypeStruct((N, F), jnp.float32),
    )(h, st, gm, bt)


def _pad8(v):
    return jnp.pad(v[None, :], ((0, 7), (0, 0)))


def _conv(x_in, xs, spmm2, spmm6, rowp, colp, zeros, dis,
          iw_c, rw0_c, b0_c, w_k, rw1_k, b1_p):
    g0p = spmm2(xs, rowp, colp, zeros)
    o1s = _kb(g0p, x_in, dis, iw_c, rw0_c, b0_c)
    gp = spmm6(o1s, rowp, colp, zeros)
    return _kc1(gp, x_in, dis, w_k, rw1_k, b1_p)


def kernel(x, edge_index, init_w1, w1, rw1, b1, g1, be1,
           init_w2, w2, rw2, b2, g2, be2):
    row = edge_index[0]
    col = edge_index[1]
    pad = E_PAD - E
    rowp = jnp.concatenate([row, jnp.zeros((pad,), jnp.int32)])
    colp = jnp.concatenate([col, jnp.full((pad,), N, jnp.int32)])
    zeros = jnp.zeros((NACC, 128), jnp.float32)
    ones = jnp.ones((1, N, 128), jnp.float32)

    def cat_kw(w3):
        return w3.transpose(1, 0, 2).reshape(F, K * F)

    iw1c = cat_kw(init_w1)
    rw01c = cat_kw(rw1[:, 0])
    b01 = _pad8(b1[:, 0, 0].transpose(0, 1).reshape(K * F))
    w1k = w1[:, 0]
    rw11k = rw1[:, 1]
    b11 = jnp.pad(b1[:, 1], ((0, 0), (0, 7), (0, 0)))
    iw2c = cat_kw(init_w2)
    rw02c = cat_kw(rw2[:, 0])
    b02 = _pad8(b2[:, 0, 0].transpose(0, 1).reshape(K * F))
    w2k = w2[:, 0]
    rw12k = rw2[:, 1]
    b12 = jnp.pad(b2[:, 1], ((0, 0), (0, 7), (0, 0)))

    spmm1 = _make_spmm(1)
    spmm2 = _make_spmm(2)
    spmm6 = _make_spmm(2 * K)

    degp = spmm1(ones, rowp, colp, zeros)
    dis, xs1 = _k1(degp, x)

    h1, st1 = _conv(x, xs1, spmm2, spmm6, rowp, colp, zeros, dis,
                    iw1c, rw01c, b01, w1k, rw11k, b11)
    y1, ys1 = _kc2_mid(h1, st1, _pad8(g1), _pad8(be1), dis)

    h2, st2 = _conv(y1, ys1, spmm2, spmm6, rowp, colp, zeros, dis,
                    iw2c, rw02c, b02, w2k, rw12k, b12)
    return _kc2_final(h2, st2, _pad8(g2), _pad8(be2))


# speedup vs baseline: 19.9126x; 1.3233x over previous
import functools

import jax
import jax.numpy as jnp
from jax import lax
from jax.experimental import pallas as pl
from jax.experimental.pallas import tpu as pltpu
from jax.experimental.pallas import tpu_sc as plsc

N = 10000
E = 160000
F = 256
K = 3
EPS = 1e-5

NSC = 2
NTILE = 16
EB = 128
E_PAD = 163840
NACC = N + 112
STR = NACC // NTILE

RB = 400
NRB = N // RB


def _make_spmm(C, gather=True):
    mesh = plsc.VectorSubcoreMesh(core_axis_name="c", subcore_axis_name="s")
    epw = E_PAD // NSC // NTILE
    nb = epw // EB

    @functools.partial(
        pl.kernel,
        mesh=mesh,
        out_type=jax.ShapeDtypeStruct((C, NSC, NACC, 128), jnp.float32),
        scratch_types=[
            pltpu.VMEM((nb, EB), jnp.int32),
            pltpu.VMEM((nb, EB), jnp.int32),
            pltpu.VMEM((EB, 128), jnp.float32),
            pltpu.VMEM((EB, 128), jnp.float32),
            pltpu.VMEM_SHARED((NACC, 128), jnp.float32),
            pltpu.SemaphoreType.DMA,
            pltpu.SemaphoreType.DMA,
        ],
    )
    def spmm(xs_hbm, rowp_hbm, colp_hbm, zeros_hbm, out_hbm,
             ridx, cidx, buf0, buf1, acc, g0, g1):
        core = lax.axis_index("c")
        sub = lax.axis_index("s")
        base_b = core * (E_PAD // NSC // EB) + sub * nb
        stripe = sub * STR
        bufs = (buf0, buf1)
        sems = (g0, g1)

        pltpu.sync_copy(rowp_hbm.at[pl.ds(base_b, nb)], ridx)
        pltpu.sync_copy(colp_hbm.at[pl.ds(base_b, nb)], cidx)

        def issue(c, b, p):
            pltpu.async_copy(xs_hbm.at[c].at[ridx.at[b]], bufs[p], sems[p])

        def wait_scatter(c, b, p):
            pltpu.make_async_copy(xs_hbm.at[c].at[ridx.at[b]],
                                  bufs[p], sems[p]).wait()
            pltpu.sync_copy(bufs[p], acc.at[cidx.at[b]], add=True)

        for c in range(C):
            pltpu.sync_copy(zeros_hbm.at[pl.ds(stripe, STR)],
                            acc.at[pl.ds(stripe, STR)])
            plsc.subcore_barrier()

            if gather:
                issue(c, 0, 0)

                def body(i, carry):
                    issue(c, 2 * i + 1, 1)
                    wait_scatter(c, 2 * i, 0)
                    issue(c, 2 * i + 2, 0)
                    wait_scatter(c, 2 * i + 1, 1)
                    return carry

                lax.fori_loop(0, (nb - 2) // 2, body, 0)
                issue(c, nb - 1, 1)
                wait_scatter(c, nb - 2, 0)
                wait_scatter(c, nb - 1, 1)
            else:
                pltpu.async_copy(xs_hbm.at[c].at[ridx.at[0]], buf0,
                                 g0).wait()

                def body(b, carry):
                    pltpu.sync_copy(buf0, acc.at[cidx.at[b]], add=True)
                    return carry

                lax.fori_loop(0, nb, body, 0)

            plsc.subcore_barrier()
            pltpu.sync_copy(acc.at[pl.ds(stripe, STR)],
                            out_hbm.at[c].at[core].at[pl.ds(stripe, STR)])

    return spmm


def _k1_body(degp_ref, x_ref, dis_ref, xs_ref):
    deg = degp_ref[0, 0] + degp_ref[0, 1]
    dis = jnp.where(deg > 0.0, lax.rsqrt(jnp.maximum(deg, 1.0)), 0.0)
    dis_ref[...] = dis
    x = x_ref[...]
    for c in range(2):
        xs_ref[c] = dis * x[:, c * 128:(c + 1) * 128]


def _k1(degp, x):
    return pl.pallas_call(
        _k1_body,
        grid=(NRB,),
        in_specs=[
            pl.BlockSpec((1, NSC, RB, 128), lambda i: (0, 0, i, 0)),
            pl.BlockSpec((RB, F), lambda i: (i, 0)),
        ],
        out_specs=[
            pl.BlockSpec((RB, 128), lambda i: (i, 0)),
            pl.BlockSpec((2, RB, 128), lambda i: (0, i, 0)),
        ],
        out_shape=[
            jax.ShapeDtypeStruct((N, 128), jnp.float32),
            jax.ShapeDtypeStruct((2, N, 128), jnp.float32),
        ],
    )(degp, x)


def _kb_body(gp_ref, x_ref, dis_ref, iw_ref, rw_ref, bias_ref, o_ref):
    g = jnp.concatenate(
        [gp_ref[c, 0] + gp_ref[c, 1] for c in range(2)], axis=1)
    d1 = dis_ref[:, :1]
    t = jnp.dot(g, iw_ref[...], preferred_element_type=jnp.float32)
    r = jnp.dot(x_ref[...], rw_ref[...], preferred_element_type=jnp.float32)
    o1 = jax.nn.relu(d1 * t + r + bias_ref[0:1, :])
    o1s = d1 * o1
    for c in range(2 * K):
        o_ref[c] = o1s[:, c * 128:(c + 1) * 128]


def _kb(gp, x, dis, iw, rw, bias):
    return pl.pallas_call(
        _kb_body,
        grid=(NRB,),
        in_specs=[
            pl.BlockSpec((2, NSC, RB, 128), lambda i: (0, 0, i, 0)),
            pl.BlockSpec((RB, F), lambda i: (i, 0)),
            pl.BlockSpec((RB, 128), lambda i: (i, 0)),
            pl.BlockSpec((F, K * F), lambda i: (0, 0)),
            pl.BlockSpec((F, K * F), lambda i: (0, 0)),
            pl.BlockSpec((8, K * F), lambda i: (0, 0)),
        ],
        out_specs=pl.BlockSpec((2 * K, RB, 128), lambda i: (0, i, 0)),
        out_shape=jax.ShapeDtypeStruct((2 * K, N, 128), jnp.float32),
    )(gp, x, dis, iw, rw, bias)


def _kc1_body(gp_ref, x_ref, dis_ref, w_ref, rw_ref, bias_ref, h_ref, st_ref):
    i = pl.program_id(0)
    d1 = dis_ref[:, :1]
    x = x_ref[...]
    acc = jnp.zeros((RB, F), jnp.float32)
    for k in range(K):
        g = jnp.concatenate(
            [gp_ref[2 * k + c, 0] + gp_ref[2 * k + c, 1] for c in range(2)],
            axis=1)
        t = jnp.dot(g, w_ref[k], preferred_element_type=jnp.float32)
        r = jnp.dot(x, rw_ref[k], preferred_element_type=jnp.float32)
        acc = acc + jax.nn.relu(d1 * t + r + bias_ref[k, 0:1, :])
    h = acc * (1.0 / K)
    h_ref[...] = h
    s1 = jnp.sum(h, axis=0, keepdims=True)
    s2 = jnp.sum(h * h, axis=0, keepdims=True)
    st = jnp.concatenate([s1, s2, jnp.zeros((6, F), jnp.float32)], axis=0)

    @pl.when(i == 0)
    def _():
        st_ref[...] = jnp.zeros_like(st_ref)

    st_ref[...] += st


def _kc1(gp, x, dis, w, rw, bias):
    return pl.pallas_call(
        _kc1_body,
        grid=(NRB,),
        in_specs=[
            pl.BlockSpec((2 * K, NSC, RB, 128), lambda i: (0, 0, i, 0)),
            pl.BlockSpec((RB, F), lambda i: (i, 0)),
            pl.BlockSpec((RB, 128), lambda i: (i, 0)),
            pl.BlockSpec((K, F, F), lambda i: (0, 0, 0)),
            pl.BlockSpec((K, F, F), lambda i: (0, 0, 0)),
            pl.BlockSpec((K, 8, F), lambda i: (0, 0, 0)),
        ],
        out_specs=[
            pl.BlockSpec((RB, F), lambda i: (i, 0)),
            pl.BlockSpec((8, F), lambda i: (0, 0)),
        ],
        out_shape=[
            jax.ShapeDtypeStruct((N, F), jnp.float32),
            jax.ShapeDtypeStruct((8, F), jnp.float32),
        ],
    )(gp, x, dis, w, rw, bias)


def _kc2_mid_body(h_ref, st_ref, gm_ref, bt_ref, dis_ref, y_ref, ys_ref):
    h = h_ref[...]
    m = st_ref[0:1, :] * (1.0 / N)
    v = st_ref[1:2, :] * (1.0 / N) - m * m
    bn = gm_ref[0:1, :] * (h - m) * lax.rsqrt(v + EPS) + bt_ref[0:1, :]
    y = jax.nn.relu(bn)
    y_ref[...] = y
    ys = dis_ref[:, :1] * y
    for c in range(2):
        ys_ref[c] = ys[:, c * 128:(c + 1) * 128]


def _kc2_mid(h, st, gm, bt, dis):
    return pl.pallas_call(
        _kc2_mid_body,
        grid=(NRB,),
        in_specs=[
            pl.BlockSpec((RB, F), lambda i: (i, 0)),
            pl.BlockSpec((8, F), lambda i: (0, 0)),
            pl.BlockSpec((8, F), lambda i: (0, 0)),
            pl.BlockSpec((8, F), lambda i: (0, 0)),
            pl.BlockSpec((RB, 128), lambda i: (i, 0)),
        ],
        out_specs=[
            pl.BlockSpec((RB, F), lambda i: (i, 0)),
            pl.BlockSpec((2, RB, 128), lambda i: (0, i, 0)),
        ],
        out_shape=[
            jax.ShapeDtypeStruct((N, F), jnp.float32),
            jax.ShapeDtypeStruct((2, N, 128), jnp.float32),
        ],
    )(h, st, gm, bt, dis)


def _kc2_final_body(h_ref, st_ref, gm_ref, bt_ref, y_ref):
    h = h_ref[...]
    m = st_ref[0:1, :] * (1.0 / N)
    v = st_ref[1:2, :] * (1.0 / N) - m * m
    bn = gm_ref[0:1, :] * (h - m) * lax.rsqrt(v + EPS) + bt_ref[0:1, :]
    y_ref[...] = jax.nn.relu(bn)


def _kc2_final(h, st, gm, bt):
    return pl.pallas_call(
        _kc2_final_body,
        grid=(NRB,),
        in_specs=[
            pl.BlockSpec((RB, F), lambda i: (i, 0)),
            pl.BlockSpec((8, F), lambda i: (0, 0)),
            pl.BlockSpec((8, F), lambda i: (0, 0)),
            pl.BlockSpec((8, F), lambda i: (0, 0)),
        ],
        out_specs=pl.BlockSpec((RB, F), lambda i: (i, 0)),
        out_shape=jax.ShapeDtypeStruct((N, F), jnp.float32),
    )(h, st, gm, bt)


def _pad8(v):
    return jnp.pad(v[None, :], ((0, 7), (0, 0)))


def _conv(x_in, xs, spmm2, spmm6, rowp, colp, zeros, dis,
          iw_c, rw0_c, b0_c, w_k, rw1_k, b1_p):
    g0p = spmm2(xs, rowp, colp, zeros)
    o1s = _kb(g0p, x_in, dis, iw_c, rw0_c, b0_c)
    gp = spmm6(o1s, rowp, colp, zeros)
    return _kc1(gp, x_in, dis, w_k, rw1_k, b1_p)


def kernel(x, edge_index, init_w1, w1, rw1, b1, g1, be1,
           init_w2, w2, rw2, b2, g2, be2):
    row = edge_index[0]
    col = edge_index[1]
    pad = E_PAD - E
    rowp = jnp.concatenate([row, jnp.zeros((pad,), jnp.int32)])
    colp = jnp.concatenate([col, jnp.full((pad,), N, jnp.int32)])
    rowp = rowp.reshape(E_PAD // EB, EB)
    colp = colp.reshape(E_PAD // EB, EB)
    zeros = jnp.zeros((NACC, 128), jnp.float32)
    ones = jnp.ones((1, N, 128), jnp.float32)

    def cat_kw(w3):
        return w3.transpose(1, 0, 2).reshape(F, K * F)

    iw1c = cat_kw(init_w1)
    rw01c = cat_kw(rw1[:, 0])
    b01 = _pad8(b1[:, 0, 0].transpose(0, 1).reshape(K * F))
    w1k = w1[:, 0]
    rw11k = rw1[:, 1]
    b11 = jnp.pad(b1[:, 1], ((0, 0), (0, 7), (0, 0)))
    iw2c = cat_kw(init_w2)
    rw02c = cat_kw(rw2[:, 0])
    b02 = _pad8(b2[:, 0, 0].transpose(0, 1).reshape(K * F))
    w2k = w2[:, 0]
    rw12k = rw2[:, 1]
    b12 = jnp.pad(b2[:, 1], ((0, 0), (0, 7), (0, 0)))

    spmm1 = _make_spmm(1, gather=False)
    spmm2 = _make_spmm(2)
    spmm6 = _make_spmm(2 * K)

    degp = spmm1(ones, rowp, colp, zeros)
    dis, xs1 = _k1(degp, x)

    h1, st1 = _conv(x, xs1, spmm2, spmm6, rowp, colp, zeros, dis,
                    iw1c, rw01c, b01, w1k, rw11k, b11)
    y1, ys1 = _kc2_mid(h1, st1, _pad8(g1), _pad8(be1), dis)

    h2, st2 = _conv(y1, ys1, spmm2, spmm6, rowp, colp, zeros, dis,
                    iw2c, rw02c, b02, w2k, rw12k, b12)
    return _kc2_final(h2, st2, _pad8(g2), _pad8(be2))
